# baseline (speedup 1.0000x reference)
import functools

import jax
import jax.numpy as jnp
from jax import lax
from jax.experimental import pallas as pl
from jax.experimental.pallas import tpu as pltpu
from jax.experimental.pallas import tpu_sc as plsc

NUM_EXP = 8
TOPK = 2
DM = 768
DFF = 3072
T = 4096
T2 = T * TOPK
CAP = int(1.25 * T * TOPK / NUM_EXP)
NSLOT = NUM_EXP * CAP
RBLK = 256
NBLK = NSLOT // RBLK + 1
NPAD = NBLK * RBLK

NC, NS = 2, 16
NW = NC * NS
CH = T2 // NW
TPT = T // NW


def _dyn_gather(vec, idx):
    return lax.gather(
        vec, idx[:, None],
        lax.GatherDimensionNumbers(offset_dims=(), collapsed_slice_dims=(0,),
                                   start_index_map=(0,)),
        (1,), mode=lax.GatherScatterMode.PROMISE_IN_BOUNDS)


def _gate_body(lg_ref, eidx_ref, g_ref, laux_ref, hist_ref):
    lg = lg_ref[:]
    iota8 = lax.broadcasted_iota(jnp.int32, lg.shape, 1)
    m = jnp.max(lg, axis=-1, keepdims=True)
    p = jnp.exp(lg - m)
    probs = p / jnp.sum(p, axis=-1, keepdims=True)
    mx1 = jnp.max(probs, axis=-1, keepdims=True)
    i1 = jnp.min(jnp.where(probs == mx1, iota8, NUM_EXP), axis=-1, keepdims=True)
    probs2 = jnp.where(iota8 == i1, -1.0, probs)
    mx2 = jnp.max(probs2, axis=-1, keepdims=True)
    i2 = jnp.min(jnp.where(probs2 == mx2, iota8, NUM_EXP), axis=-1, keepdims=True)
    pv1 = jnp.sum(jnp.where(iota8 == i1, probs, 0.0), axis=-1, keepdims=True)
    pv2 = jnp.sum(jnp.where(iota8 == i2, probs, 0.0), axis=-1, keepdims=True)
    den = pv1 + pv2
    eidx_ref[:] = jnp.concatenate([i1, i2], axis=1)
    g_ref[:] = jnp.concatenate([pv1 / den, pv2 / den], axis=1)
    me = jnp.sum(probs, axis=0) / T
    ce = jnp.sum(jnp.where(iota8 == i1, 1.0, 0.0), axis=0) / T
    laux_ref[:] = (jnp.sum(me * ce) * NUM_EXP)[None, None]
    iota16 = lax.broadcasted_iota(jnp.int32, (T, 16), 1)
    oh = (jnp.where(iota16 == i1, 1, 0) + jnp.where(iota16 == i2, 1, 0))
    hist_ref[:] = jnp.sum(oh.reshape(NW, TPT, 16), axis=1)


def _gate(logits):
    return pl.pallas_call(
        _gate_body,
        out_shape=[jax.ShapeDtypeStruct((T, 2), jnp.int32),
                   jax.ShapeDtypeStruct((T, 2), jnp.float32),
                   jax.ShapeDtypeStruct((1, 1), jnp.float32),
                   jax.ShapeDtypeStruct((NW, 16), jnp.int32)],
    )(logits)


NF = 2


def _ffn_body(cnt_ref, xb, w1, w2, ob):
    i = pl.program_id(0)
    f = pl.program_id(1)
    e = jnp.minimum(i // (CAP // RBLK), NUM_EXP - 1)
    live = jnp.logical_and(i < NBLK - 1,
                           lax.rem(i, CAP // RBLK) * RBLK < cnt_ref[e])

    @pl.when(live)
    def _():
        h = jnp.maximum(jnp.dot(xb[:], w1[0], preferred_element_type=jnp.float32), 0.0)
        yp = jnp.dot(h, w2[0], preferred_element_type=jnp.float32)
        ob[:] = jnp.where(f == 0, yp, ob[:] + yp)

    @pl.when(jnp.logical_not(live))
    def _():
        ob[:] = jnp.zeros_like(ob)


def _ffn(counts, buf, w1, w2):
    emap = lambda i, f, c: jnp.minimum(i // (CAP // RBLK), NUM_EXP - 1)
    return pl.pallas_call(
        _ffn_body,
        grid_spec=pltpu.PrefetchScalarGridSpec(
            num_scalar_prefetch=1,
            grid=(NBLK, NF),
            in_specs=[pl.BlockSpec((RBLK, DM), lambda i, f, c: (i, 0)),
                      pl.BlockSpec((1, DM, DFF // NF),
                                   lambda i, f, c: (emap(i, f, c), 0, f)),
                      pl.BlockSpec((1, DFF // NF, DM),
                                   lambda i, f, c: (emap(i, f, c), f, 0)),
                      ],
            out_specs=pl.BlockSpec((RBLK, DM), lambda i, f, c: (i, 0))),
        out_shape=jax.ShapeDtypeStruct((NPAD, DM), jnp.float32),
    )(counts, buf, w1, w2)


@functools.cache
def _sc_kernels():
    mesh = plsc.VectorSubcoreMesh(core_axis_name="c", subcore_axis_name="s",
                                  num_cores=NC, num_subcores=NS)

    def _wid():
        return lax.axis_index("s") * NC + lax.axis_index("c")

    @functools.partial(
        pl.kernel,
        out_type=[jax.ShapeDtypeStruct((2, T), jnp.int32),
                  jax.ShapeDtypeStruct((16,), jnp.int32),
                  jax.ShapeDtypeStruct((NPAD, DM), jnp.float32)],
        mesh=mesh,
        compiler_params=pltpu.CompilerParams(needs_layout_passes=False),
        scratch_types=[pltpu.VMEM((CH,), jnp.int32),
                       pltpu.VMEM((NW, 16), jnp.int32),
                       pltpu.VMEM((2, TPT), jnp.int32),
                       pltpu.VMEM((16,), jnp.int32),
                       pltpu.VMEM((TPT, DM), jnp.float32),
                       pltpu.SemaphoreType.DMA,
                       pltpu.SemaphoreType.DMA,
                       pltpu.SemaphoreType.DMA])
    def _route_k(e_hbm, hist_hbm, x_hbm, dest_hbm, cnt_hbm, buf_hbm,
                 ev, hv, dv2, cv, xv, sx, se, so):
        w = _wid()
        cpx = pltpu.async_copy(x_hbm.at[pl.ds(w * TPT, TPT)], xv, sx)
        pltpu.sync_copy(e_hbm.at[pl.ds(w * CH, CH)], ev)
        pltpu.sync_copy(hist_hbm, hv)
        iota = lax.iota(jnp.int32, 16)
        zero = jnp.zeros((16,), jnp.int32)
        base, total = zero, zero
        for wp in range(NW):
            row = hv[wp]
            base = base + jnp.where(jnp.full((16,), wp, jnp.int32) < w, row, 0)
            total = total + row

        @pl.when(w == 0)
        def _():
            cv[...] = jnp.minimum(total, CAP)
            pltpu.sync_copy(cv, cnt_hbm)

        carry = base
        for j in range(CH // 16):
            v = ev[pl.ds(j * 16, 16)]
            prior, add = zero, zero
            for e in range(NUM_EXP):
                msk = (v == e)
                cs = plsc.cumsum(jnp.where(msk, 1, 0))
                prior = prior + jnp.where(msk, cs - 1, 0)
                pc = plsc.all_reduce_population_count(msk)
                add = add + jnp.where(iota == e, pc, 0)
            pos = _dyn_gather(carry, v) + prior
            keep = pos < CAP
            destv = jnp.where(keep, v * CAP + pos, NSLOT)
            lp = j * 16 + iota
            plsc.store_scatter(dv2, [lp & 1, lp >> 1], destv)
            carry = carry + add
        pltpu.sync_copy(dv2, dest_hbm.at[:, pl.ds(w * TPT, TPT)])
        cpx.wait()
        ce = pltpu.async_copy(xv, buf_hbm.at[dv2.at[0]], se)
        co = pltpu.async_copy(xv, buf_hbm.at[dv2.at[1]], so)
        ce.wait()
        co.wait()

    CCH = 32

    @functools.partial(
        pl.kernel,
        out_type=jax.ShapeDtypeStruct((T, DM), jnp.float32),
        mesh=mesh,
        compiler_params=pltpu.CompilerParams(needs_layout_passes=False),
        scratch_types=[pltpu.VMEM((TPT,), jnp.int32),
                       pltpu.VMEM((TPT,), jnp.int32),
                       pltpu.VMEM((TPT + 16,), jnp.float32),
                       pltpu.VMEM((TPT + 16,), jnp.float32),
                       [pltpu.VMEM((CCH, DM), jnp.float32) for _ in range(2)],
                       [pltpu.VMEM((CCH, DM), jnp.float32) for _ in range(2)],
                       pltpu.VMEM((CCH, DM), jnp.float32),
                       [pltpu.SemaphoreType.DMA for _ in range(2)],
                       [pltpu.SemaphoreType.DMA for _ in range(2)],
                       pltpu.SemaphoreType.DMA])
    def _comb_k(y_hbm, dest_hbm, g_hbm, out_hbm, de, do, ge, go, re, ro, ov,
                semE, semO, semS):
        w = _wid()
        pltpu.sync_copy(dest_hbm.at[0, pl.ds(w * TPT, TPT)], de)
        pltpu.sync_copy(dest_hbm.at[1, pl.ds(w * TPT, TPT)], do)
        pltpu.sync_copy(g_hbm.at[0, pl.ds(w * TPT, TPT)], ge.at[pl.ds(0, TPT)])
        pltpu.sync_copy(g_hbm.at[1, pl.ds(w * TPT, TPT)], go.at[pl.ds(0, TPT)])
        nch = TPT // CCH
        stores = []
        cps = [(pltpu.async_copy(y_hbm.at[de.at[pl.ds(0, CCH)]], re[0], semE[0]),
                pltpu.async_copy(y_hbm.at[do.at[pl.ds(0, CCH)]], ro[0], semO[0]))]
        for c in range(nch):
            if c + 1 < nch:
                b = (c + 1) % 2
                cps.append((
                    pltpu.async_copy(y_hbm.at[de.at[pl.ds((c + 1) * CCH, CCH)]],
                                     re[b], semE[b]),
                    pltpu.async_copy(y_hbm.at[do.at[pl.ds((c + 1) * CCH, CCH)]],
                                     ro[b], semO[b])))
            cps[c][0].wait()
            cps[c][1].wait()
            rec, roc, ovc = re[c % 2], ro[c % 2], ov
            if c >= 1:
                stores[c - 1].wait()

            def body(i, _):
                for u in range(2):
                    r = 2 * i + u
                    a = ge[pl.ds(c * CCH + r, 16)][0]
                    b2 = go[pl.ds(c * CCH + r, 16)][0]
                    for lc in range(DM // 16):
                        sl = pl.ds(lc * 16, 16)
                        ovc[r, sl] = a * rec[r, sl] + b2 * roc[r, sl]
                return 0

            lax.fori_loop(0, CCH // 2, body, 0)
            stores.append(pltpu.async_copy(
                ovc, out_hbm.at[pl.ds(w * TPT + c * CCH, CCH)], semS))
        stores[-1].wait()

    return _route_k, _comb_k


def kernel(x, Wg, W1, W2):
    route_k, comb_k = _sc_kernels()
    logits = x @ Wg
    eidx, g, laux, hist = _gate(logits)
    e_flat = eidx.reshape(-1)
    dest2, counts, buf = route_k(e_flat, hist, x)
    y_e = _ffn(counts, buf, W1, W2)
    y = comb_k(y_e, dest2, g.T)
    return y, laux[0, 0]


# speedup vs baseline: 1.4158x; 1.4158x over previous
import functools

import jax
import jax.numpy as jnp
from jax import lax
from jax.experimental import pallas as pl
from jax.experimental.pallas import tpu as pltpu
from jax.experimental.pallas import tpu_sc as plsc

NUM_EXP = 8
TOPK = 2
DM = 768
DFF = 3072
T = 4096
T2 = T * TOPK
CAP = int(1.25 * T * TOPK / NUM_EXP)
NSLOT = NUM_EXP * CAP
RBLK = 256
NBLK = NSLOT // RBLK + 1
NPAD = NBLK * RBLK

NC, NS = 2, 16
NW = NC * NS
CH = T2 // NW
TPT = T // NW


def _dyn_gather(vec, idx):
    return lax.gather(
        vec, idx[:, None],
        lax.GatherDimensionNumbers(offset_dims=(), collapsed_slice_dims=(0,),
                                   start_index_map=(0,)),
        (1,), mode=lax.GatherScatterMode.PROMISE_IN_BOUNDS)


def _gate_body(lg_ref, eidx_ref, g_ref, laux_ref, hist_ref):
    lg = lg_ref[:]
    iota8 = lax.broadcasted_iota(jnp.int32, lg.shape, 1)
    m = jnp.max(lg, axis=-1, keepdims=True)
    p = jnp.exp(lg - m)
    probs = p / jnp.sum(p, axis=-1, keepdims=True)
    mx1 = jnp.max(probs, axis=-1, keepdims=True)
    i1 = jnp.min(jnp.where(probs == mx1, iota8, NUM_EXP), axis=-1, keepdims=True)
    probs2 = jnp.where(iota8 == i1, -1.0, probs)
    mx2 = jnp.max(probs2, axis=-1, keepdims=True)
    i2 = jnp.min(jnp.where(probs2 == mx2, iota8, NUM_EXP), axis=-1, keepdims=True)
    pv1 = jnp.sum(jnp.where(iota8 == i1, probs, 0.0), axis=-1, keepdims=True)
    pv2 = jnp.sum(jnp.where(iota8 == i2, probs, 0.0), axis=-1, keepdims=True)
    den = pv1 + pv2
    eidx_ref[:] = jnp.concatenate([i1, i2], axis=1)
    g_ref[:] = jnp.concatenate([pv1 / den, pv2 / den], axis=1)
    me = jnp.sum(probs, axis=0) / T
    ce = jnp.sum(jnp.where(iota8 == i1, 1.0, 0.0), axis=0) / T
    laux_ref[:] = (jnp.sum(me * ce) * NUM_EXP)[None, None]
    iota16 = lax.broadcasted_iota(jnp.int32, (T, 16), 1)
    oh = (jnp.where(iota16 == i1, 1, 0) + jnp.where(iota16 == i2, 1, 0))
    hist_ref[:] = jnp.sum(oh.reshape(NW, TPT, 16), axis=1)


def _gate(logits):
    return pl.pallas_call(
        _gate_body,
        out_shape=[jax.ShapeDtypeStruct((T, 2), jnp.int32),
                   jax.ShapeDtypeStruct((T, 2), jnp.float32),
                   jax.ShapeDtypeStruct((1, 1), jnp.float32),
                   jax.ShapeDtypeStruct((NW, 16), jnp.int32)],
    )(logits)


def _ffn_body(cnt_ref, xb, w1, w2, ob):
    i = pl.program_id(0)
    e = jnp.minimum(i // (CAP // RBLK), NUM_EXP - 1)
    live = jnp.logical_and(i < NBLK - 1,
                           lax.rem(i, CAP // RBLK) * RBLK < cnt_ref[e])

    @pl.when(live)
    def _():
        h = jnp.maximum(jnp.dot(xb[:], w1[0], preferred_element_type=jnp.float32), 0.0)
        ob[:] = jnp.dot(h, w2[0], preferred_element_type=jnp.float32)

    @pl.when(jnp.logical_not(live))
    def _():
        ob[:] = jnp.zeros_like(ob)


def _ffn(counts, buf, w1, w2):
    eix = lambda i, c: (jnp.minimum(i // (CAP // RBLK), NUM_EXP - 1), 0, 0)
    return pl.pallas_call(
        _ffn_body,
        grid_spec=pltpu.PrefetchScalarGridSpec(
            num_scalar_prefetch=1,
            grid=(NBLK,),
            in_specs=[pl.BlockSpec((RBLK, DM), lambda i, c: (i, 0)),
                      pl.BlockSpec((1, DM, DFF), eix),
                      pl.BlockSpec((1, DFF, DM), eix)],
            out_specs=pl.BlockSpec((RBLK, DM), lambda i, c: (i, 0))),
        out_shape=jax.ShapeDtypeStruct((NPAD, DM), jnp.float32),
    )(counts, buf, w1, w2)


@functools.cache
def _sc_kernels():
    mesh = plsc.VectorSubcoreMesh(core_axis_name="c", subcore_axis_name="s",
                                  num_cores=NC, num_subcores=NS)

    def _wid():
        return lax.axis_index("s") * NC + lax.axis_index("c")

    @functools.partial(
        pl.kernel,
        out_type=[jax.ShapeDtypeStruct((2, T), jnp.int32),
                  jax.ShapeDtypeStruct((16,), jnp.int32),
                  jax.ShapeDtypeStruct((NPAD, DM), jnp.float32)],
        mesh=mesh,
        compiler_params=pltpu.CompilerParams(needs_layout_passes=False),
        scratch_types=[pltpu.VMEM((CH,), jnp.int32),
                       pltpu.VMEM((NW, 16), jnp.int32),
                       pltpu.VMEM((2, TPT), jnp.int32),
                       pltpu.VMEM((16,), jnp.int32),
                       pltpu.VMEM((TPT, DM), jnp.float32),
                       pltpu.SemaphoreType.DMA,
                       pltpu.SemaphoreType.DMA,
                       pltpu.SemaphoreType.DMA])
    def _route_k(e_hbm, hist_hbm, x_hbm, dest_hbm, cnt_hbm, buf_hbm,
                 ev, hv, dv2, cv, xv, sx, se, so):
        w = _wid()
        cpx = pltpu.async_copy(x_hbm.at[pl.ds(w * TPT, TPT)], xv, sx)
        pltpu.sync_copy(e_hbm.at[pl.ds(w * CH, CH)], ev)
        pltpu.sync_copy(hist_hbm, hv)
        iota = lax.iota(jnp.int32, 16)
        zero = jnp.zeros((16,), jnp.int32)
        base, total = zero, zero
        for wp in range(NW):
            row = hv[wp]
            base = base + jnp.where(jnp.full((16,), wp, jnp.int32) < w, row, 0)
            total = total + row

        @pl.when(w == 0)
        def _():
            cv[...] = jnp.minimum(total, CAP)
            pltpu.sync_copy(cv, cnt_hbm)

        carry = base
        for j in range(CH // 16):
            v = ev[pl.ds(j * 16, 16)]
            prior, add = zero, zero
            for e in range(NUM_EXP):
                msk = (v == e)
                cs = plsc.cumsum(jnp.where(msk, 1, 0))
                prior = prior + jnp.where(msk, cs - 1, 0)
                pc = plsc.all_reduce_population_count(msk)
                add = add + jnp.where(iota == e, pc, 0)
            pos = _dyn_gather(carry, v) + prior
            keep = pos < CAP
            destv = jnp.where(keep, v * CAP + pos, NSLOT)
            lp = j * 16 + iota
            plsc.store_scatter(dv2, [lp & 1, lp >> 1], destv)
            carry = carry + add
        pltpu.sync_copy(dv2, dest_hbm.at[:, pl.ds(w * TPT, TPT)])
        cpx.wait()
        ce = pltpu.async_copy(xv, buf_hbm.at[dv2.at[0]], se)
        co = pltpu.async_copy(xv, buf_hbm.at[dv2.at[1]], so)
        ce.wait()
        co.wait()

    CCH = 32

    @functools.partial(
        pl.kernel,
        out_type=jax.ShapeDtypeStruct((T, DM), jnp.float32),
        mesh=mesh,
        compiler_params=pltpu.CompilerParams(needs_layout_passes=False),
        scratch_types=[pltpu.VMEM((TPT,), jnp.int32),
                       pltpu.VMEM((TPT,), jnp.int32),
                       pltpu.VMEM((TPT + 16,), jnp.float32),
                       pltpu.VMEM((TPT + 16,), jnp.float32),
                       [pltpu.VMEM((CCH, DM), jnp.float32) for _ in range(2)],
                       [pltpu.VMEM((CCH, DM), jnp.float32) for _ in range(2)],
                       pltpu.VMEM((CCH, DM), jnp.float32),
                       [pltpu.SemaphoreType.DMA for _ in range(2)],
                       [pltpu.SemaphoreType.DMA for _ in range(2)],
                       pltpu.SemaphoreType.DMA])
    def _comb_k(y_hbm, dest_hbm, g_hbm, out_hbm, de, do, ge, go, re, ro, ov,
                semE, semO, semS):
        w = _wid()
        pltpu.sync_copy(dest_hbm.at[0, pl.ds(w * TPT, TPT)], de)
        pltpu.sync_copy(dest_hbm.at[1, pl.ds(w * TPT, TPT)], do)
        pltpu.sync_copy(g_hbm.at[0, pl.ds(w * TPT, TPT)], ge.at[pl.ds(0, TPT)])
        pltpu.sync_copy(g_hbm.at[1, pl.ds(w * TPT, TPT)], go.at[pl.ds(0, TPT)])
        nch = TPT // CCH
        stores = []
        cps = [(pltpu.async_copy(y_hbm.at[de.at[pl.ds(0, CCH)]], re[0], semE[0]),
                pltpu.async_copy(y_hbm.at[do.at[pl.ds(0, CCH)]], ro[0], semO[0]))]
        for c in range(nch):
            if c + 1 < nch:
                b = (c + 1) % 2
                cps.append((
                    pltpu.async_copy(y_hbm.at[de.at[pl.ds((c + 1) * CCH, CCH)]],
                                     re[b], semE[b]),
                    pltpu.async_copy(y_hbm.at[do.at[pl.ds((c + 1) * CCH, CCH)]],
                                     ro[b], semO[b])))
            cps[c][0].wait()
            cps[c][1].wait()
            rec, roc, ovc = re[c % 2], ro[c % 2], ov
            if c >= 1:
                stores[c - 1].wait()

            def body(i, _):
                for u in range(2):
                    r = 2 * i + u
                    a = ge[pl.ds(c * CCH + r, 16)][0]
                    b2 = go[pl.ds(c * CCH + r, 16)][0]
                    for lc in range(DM // 16):
                        sl = pl.ds(lc * 16, 16)
                        ovc[r, sl] = a * rec[r, sl] + b2 * roc[r, sl]
                return 0

            lax.fori_loop(0, CCH // 2, body, 0)
            stores.append(pltpu.async_copy(
                ovc, out_hbm.at[pl.ds(w * TPT + c * CCH, CCH)], semS))
        stores[-1].wait()

    return _route_k, _comb_k


def kernel(x, Wg, W1, W2):
    route_k, comb_k = _sc_kernels()
    logits = x @ Wg
    eidx, g, laux, hist = _gate(logits)
    e_flat = eidx.reshape(-1)
    dest2, counts, buf = route_k(e_flat, hist, x)
    y_e = _ffn(counts, buf, W1, W2)
    y = comb_k(y_e, dest2, g.T)
    return y, laux[0, 0]


# speedup vs baseline: 1.4991x; 1.0588x over previous
import functools

import jax
import jax.numpy as jnp
from jax import lax
from jax.experimental import pallas as pl
from jax.experimental.pallas import tpu as pltpu
from jax.experimental.pallas import tpu_sc as plsc

NUM_EXP = 8
TOPK = 2
DM = 768
DFF = 3072
T = 4096
T2 = T * TOPK
CAP = int(1.25 * T * TOPK / NUM_EXP)
NSLOT = NUM_EXP * CAP
RBLK = 256
NBLK = NSLOT // RBLK + 1
NPAD = NBLK * RBLK

NC, NS = 2, 16
NW = NC * NS
CH = T2 // NW
TPT = T // NW


def _dyn_gather(vec, idx):
    return lax.gather(
        vec, idx[:, None],
        lax.GatherDimensionNumbers(offset_dims=(), collapsed_slice_dims=(0,),
                                   start_index_map=(0,)),
        (1,), mode=lax.GatherScatterMode.PROMISE_IN_BOUNDS)


def _gate_body(lg_ref, eidx_ref, g_ref, laux_ref, hist_ref):
    lg = lg_ref[:]
    iota8 = lax.broadcasted_iota(jnp.int32, lg.shape, 0)
    m = jnp.max(lg, axis=0, keepdims=True)
    p = jnp.exp(lg - m)
    probs = p / jnp.sum(p, axis=0, keepdims=True)
    mx1 = jnp.max(probs, axis=0, keepdims=True)
    i1 = jnp.min(jnp.where(probs == mx1, iota8, NUM_EXP), axis=0, keepdims=True)
    probs2 = jnp.where(iota8 == i1, -1.0, probs)
    mx2 = jnp.max(probs2, axis=0, keepdims=True)
    i2 = jnp.min(jnp.where(probs2 == mx2, iota8, NUM_EXP), axis=0, keepdims=True)
    pv1 = jnp.sum(jnp.where(iota8 == i1, probs, 0.0), axis=0, keepdims=True)
    pv2 = jnp.sum(jnp.where(iota8 == i2, probs, 0.0), axis=0, keepdims=True)
    den = pv1 + pv2
    eidx_ref[:] = jnp.concatenate([i1, i2], axis=0)
    g_ref[:] = jnp.concatenate([pv1 / den, pv2 / den], axis=0)
    oh1 = jnp.where(iota8 == i1, 1.0, 0.0)
    me = jnp.sum(probs, axis=1) / T
    ce = jnp.sum(oh1, axis=1) / T
    laux_ref[:] = (jnp.sum(me * ce) * NUM_EXP)[None, None]
    oh = oh1 + jnp.where(iota8 == i2, 1.0, 0.0)
    hist_ref[:] = jnp.sum(oh.reshape(NUM_EXP, NW, TPT), axis=2).astype(jnp.int32)


def _gate(logits_t):
    return pl.pallas_call(
        _gate_body,
        out_shape=[jax.ShapeDtypeStruct((2, T), jnp.int32),
                   jax.ShapeDtypeStruct((2, T), jnp.float32),
                   jax.ShapeDtypeStruct((1, 1), jnp.float32),
                   jax.ShapeDtypeStruct((NUM_EXP, NW), jnp.int32)],
    )(logits_t)


def _ffn_body(cnt_ref, xb, w1, w2, ob):
    i = pl.program_id(0)
    e = jnp.minimum(i // (CAP // RBLK), NUM_EXP - 1)
    live = jnp.logical_and(i < NBLK - 1,
                           lax.rem(i, CAP // RBLK) * RBLK < cnt_ref[e])

    @pl.when(live)
    def _():
        h = jnp.maximum(jnp.dot(xb[:], w1[0], preferred_element_type=jnp.float32), 0.0)
        ob[:] = jnp.dot(h, w2[0], preferred_element_type=jnp.float32)

    @pl.when(jnp.logical_not(live))
    def _():
        ob[:] = jnp.zeros_like(ob)


def _ffn(counts, buf, w1, w2):
    eix = lambda i, c: (jnp.minimum(i // (CAP // RBLK), NUM_EXP - 1), 0, 0)
    return pl.pallas_call(
        _ffn_body,
        grid_spec=pltpu.PrefetchScalarGridSpec(
            num_scalar_prefetch=1,
            grid=(NBLK,),
            in_specs=[pl.BlockSpec((RBLK, DM), lambda i, c: (i, 0)),
                      pl.BlockSpec((1, DM, DFF), eix),
                      pl.BlockSpec((1, DFF, DM), eix)],
            out_specs=pl.BlockSpec((RBLK, DM), lambda i, c: (i, 0))),
        out_shape=jax.ShapeDtypeStruct((NPAD, DM), jnp.float32),
    )(counts, buf, w1, w2)


@functools.cache
def _sc_kernels():
    mesh = plsc.VectorSubcoreMesh(core_axis_name="c", subcore_axis_name="s",
                                  num_cores=NC, num_subcores=NS)

    def _wid():
        return lax.axis_index("s") * NC + lax.axis_index("c")

    @functools.partial(
        pl.kernel,
        out_type=[jax.ShapeDtypeStruct((2, T), jnp.int32),
                  jax.ShapeDtypeStruct((16,), jnp.int32),
                  jax.ShapeDtypeStruct((NPAD, DM), jnp.float32)],
        mesh=mesh,
        compiler_params=pltpu.CompilerParams(needs_layout_passes=False),
        scratch_types=[pltpu.VMEM((CH,), jnp.int32),
                       pltpu.VMEM((2, TPT), jnp.int32),
                       pltpu.VMEM((NUM_EXP, NW), jnp.int32),
                       pltpu.VMEM((2, TPT), jnp.int32),
                       pltpu.VMEM((16,), jnp.int32),
                       pltpu.VMEM((TPT, DM), jnp.float32),
                       pltpu.SemaphoreType.DMA,
                       pltpu.SemaphoreType.DMA,
                       pltpu.SemaphoreType.DMA])
    def _route_k(e_hbm, hist_hbm, x_hbm, dest_hbm, cnt_hbm, buf_hbm,
                 ev, et, hv, dv2, cv, xv, sx, se, so):
        w = _wid()
        cpx = pltpu.async_copy(x_hbm.at[pl.ds(w * TPT, TPT)], xv, sx)
        pltpu.sync_copy(e_hbm.at[:, pl.ds(w * TPT, TPT)], et)
        pltpu.sync_copy(hist_hbm, hv)
        iota = lax.iota(jnp.int32, 16)
        zero = jnp.zeros((16,), jnp.int32)
        for j in range(TPT // 16):
            lp = 32 * j + 2 * iota
            plsc.store_scatter(ev, [lp], et[0, pl.ds(j * 16, 16)])
            plsc.store_scatter(ev, [lp + 1], et[1, pl.ds(j * 16, 16)])
        base, total = zero, zero
        for e in range(NUM_EXP):
            h0 = hv[e, pl.ds(0, 16)]
            h1 = hv[e, pl.ds(16, 16)]
            be = (jnp.sum(jnp.where(iota < w, h0, 0))
                  + jnp.sum(jnp.where(iota + 16 < w, h1, 0)))
            te = jnp.sum(h0) + jnp.sum(h1)
            base = base + jnp.where(iota == e, be, 0)
            total = total + jnp.where(iota == e, te, 0)

        @pl.when(w == 0)
        def _():
            cv[...] = jnp.minimum(total, CAP)
            pltpu.sync_copy(cv, cnt_hbm)

        carry = base
        for j in range(CH // 16):
            v = ev[pl.ds(j * 16, 16)]
            prior, add = zero, zero
            for e in range(NUM_EXP):
                msk = (v == e)
                cs = plsc.cumsum(jnp.where(msk, 1, 0))
                prior = prior + jnp.where(msk, cs - 1, 0)
                pc = plsc.all_reduce_population_count(msk)
                add = add + jnp.where(iota == e, pc, 0)
            pos = _dyn_gather(carry, v) + prior
            keep = pos < CAP
            destv = jnp.where(keep, v * CAP + pos, NSLOT)
            lp = j * 16 + iota
            plsc.store_scatter(dv2, [lp & 1, lp >> 1], destv)
            carry = carry + add
        pltpu.sync_copy(dv2, dest_hbm.at[:, pl.ds(w * TPT, TPT)])
        cpx.wait()
        ce = pltpu.async_copy(xv, buf_hbm.at[dv2.at[0]], se)
        co = pltpu.async_copy(xv, buf_hbm.at[dv2.at[1]], so)
        ce.wait()
        co.wait()

    CCH = 32

    @functools.partial(
        pl.kernel,
        out_type=jax.ShapeDtypeStruct((T, DM), jnp.float32),
        mesh=mesh,
        compiler_params=pltpu.CompilerParams(needs_layout_passes=False),
        scratch_types=[pltpu.VMEM((TPT,), jnp.int32),
                       pltpu.VMEM((TPT,), jnp.int32),
                       pltpu.VMEM((TPT + 16,), jnp.float32),
                       pltpu.VMEM((TPT + 16,), jnp.float32),
                       [pltpu.VMEM((CCH, DM), jnp.float32) for _ in range(2)],
                       [pltpu.VMEM((CCH, DM), jnp.float32) for _ in range(2)],
                       [pltpu.SemaphoreType.DMA for _ in range(2)],
                       [pltpu.SemaphoreType.DMA for _ in range(2)]])
    def _comb_k(y_hbm, dest_hbm, g_hbm, out_hbm, de, do, ge, go, re, ro,
                semE, semO):
        w = _wid()
        pltpu.sync_copy(dest_hbm.at[0, pl.ds(w * TPT, TPT)], de)
        pltpu.sync_copy(dest_hbm.at[1, pl.ds(w * TPT, TPT)], do)
        pltpu.sync_copy(g_hbm.at[0, pl.ds(w * TPT, TPT)], ge.at[pl.ds(0, TPT)])
        pltpu.sync_copy(g_hbm.at[1, pl.ds(w * TPT, TPT)], go.at[pl.ds(0, TPT)])
        nch = TPT // CCH
        cps = [(pltpu.async_copy(y_hbm.at[de.at[pl.ds(0, CCH)]], re[0], semE[0]),
                pltpu.async_copy(y_hbm.at[do.at[pl.ds(0, CCH)]], ro[0], semO[0]))]
        for c in range(nch):
            if c + 1 < nch:
                b = (c + 1) % 2
                cps.append((
                    pltpu.async_copy(y_hbm.at[de.at[pl.ds((c + 1) * CCH, CCH)]],
                                     re[b], semE[b]),
                    pltpu.async_copy(y_hbm.at[do.at[pl.ds((c + 1) * CCH, CCH)]],
                                     ro[b], semO[b])))
            cps[c][0].wait()
            cps[c][1].wait()
            rec, roc = re[c % 2], ro[c % 2]

            def body(i, _):
                a = ge[pl.ds(c * CCH + i, 16)][0]
                b2 = go[pl.ds(c * CCH + i, 16)][0]
                for lc in range(DM // 16):
                    sl = pl.ds(lc * 16, 16)
                    rec[i, sl] = a * rec[i, sl] + b2 * roc[i, sl]
                return 0

            lax.fori_loop(0, CCH, body, 0)
            pltpu.sync_copy(rec, out_hbm.at[pl.ds(w * TPT + c * CCH, CCH)])

    return _route_k, _comb_k


def kernel(x, Wg, W1, W2):
    route_k, comb_k = _sc_kernels()
    logits_t = jnp.transpose(x @ Wg)
    eidx2, g2, laux, hist = _gate(logits_t)
    dest2, counts, buf = route_k(eidx2, hist, x)
    y_e = _ffn(counts, buf, W1, W2)
    y = comb_k(y_e, dest2, g2)
    return y, laux[0, 0]
